# Initial kernel scaffold; baseline (speedup 1.0000x reference)
import functools

import jax
import jax.numpy as jnp
from jax import lax
from jax.experimental import pallas as pl
from jax.experimental.pallas import tpu as pltpu
from jax.experimental.pallas import tpu_sc as plsc

S = 8192
D = 4096
L = 16
_info = plsc.get_sparse_core_info()
NC = _info.num_cores
NS = _info.num_subcores
NW = NC * NS
SLOTS_W = S // NW
CSIZE = 16
NCHUNK = SLOTS_W // CSIZE


def _body(x_hbm, m_hbm, out_hbm, mask_v, idx_v, rows_v, gsem, ssem):
    wid = lax.axis_index("s") * NC + lax.axis_index("c")
    lo = wid * SLOTS_W

    pltpu.sync_copy(m_hbm, mask_v)

    def p1(i, acc):
        return acc + mask_v[pl.ds(i * L, L)]

    acc = lax.fori_loop(0, S // L, p1, jnp.zeros((L,), jnp.int32))
    n_true = jnp.sum(acc)

    lane = lax.iota(jnp.int32, L)

    def p2(i, c1):
        m = mask_v[pl.ds(i * L, L)]
        cs = plsc.cumsum(m)
        g = i * L + lane
        rank = jnp.where(m == 1, c1 + cs - 1, n_true + g - c1 - cs)
        r = rank - lo
        ok = (r >= 0) & (r < SLOTS_W)
        rs = jnp.where(ok, r, 0)
        plsc.store_scatter(idx_v, [rs >> 4, rs & 15], g, mask=ok)
        return c1 + jnp.sum(m)

    lax.fori_loop(0, S // L, p2, jnp.int32(0))

    def mv(j, _):
        idx = idx_v.at[j]
        pltpu.async_copy(x_hbm.at[idx], rows_v, gsem).wait()
        pltpu.async_copy(rows_v, out_hbm.at[idx], ssem).wait()
        return 0

    lax.fori_loop(0, NCHUNK, mv, 0)


_routed = functools.partial(
    pl.kernel,
    mesh=plsc.VectorSubcoreMesh(core_axis_name="c", subcore_axis_name="s"),
    out_type=jax.ShapeDtypeStruct((S, D), jnp.float32),
    scratch_types=[
        pltpu.VMEM((S,), jnp.int32),
        pltpu.VMEM((NCHUNK, CSIZE), jnp.int32),
        pltpu.VMEM((CSIZE, D), jnp.float32),
        pltpu.SemaphoreType.DMA,
        pltpu.SemaphoreType.DMA,
    ],
)(_routed_body := _body)


def kernel(x, masks):
    out = _routed(x[0], masks[0].astype(jnp.int32))
    return out[None]


# speedup vs baseline: 11.7792x; 11.7792x over previous
import functools

import jax
import jax.numpy as jnp
from jax import lax
from jax.experimental import pallas as pl
from jax.experimental.pallas import tpu as pltpu
from jax.experimental.pallas import tpu_sc as plsc

S = 8192
D = 4096
L = 16
_info = plsc.get_sparse_core_info()
NC = _info.num_cores
NS = _info.num_subcores
NW = NC * NS
SLOTS_W = S // NW
CSIZE = 16
NCHUNK = SLOTS_W // CSIZE


def _body(x_hbm, m_hbm, out_hbm, mask_v, idx_v, rows_v, gsem, ssem):
    wid = lax.axis_index("s") * NC + lax.axis_index("c")
    lo = wid * SLOTS_W

    pltpu.sync_copy(m_hbm, mask_v)

    def p1(i, acc):
        return acc + mask_v[pl.ds(i * L, L)]

    acc = lax.fori_loop(0, S // L, p1, jnp.zeros((L,), jnp.int32))
    n_true = jnp.sum(acc)

    lane = lax.iota(jnp.int32, L)

    def p2(i, c1):
        m = mask_v[pl.ds(i * L, L)]
        cs = plsc.cumsum(m)
        g = i * L + lane
        rank = jnp.where(m == 1, c1 + cs - 1, n_true + g - c1 - cs)
        r = rank - lo
        ok = (r >= 0) & (r < SLOTS_W)
        rs = jnp.where(ok, r, 0)
        plsc.store_scatter(idx_v, [rs >> 4, rs & 15], g, mask=ok)
        return c1 + jnp.sum(m)

    lax.fori_loop(0, S // L, p2, jnp.int32(0))

    def mv(j, _):
        idx = idx_v.at[j]
        pltpu.async_copy(x_hbm.at[idx], rows_v, gsem).wait()
        pltpu.async_copy(rows_v, out_hbm.at[idx], ssem).wait()
        return 0

    lax.fori_loop(0, NCHUNK, mv, 0)


_routed = functools.partial(
    pl.kernel,
    mesh=plsc.VectorSubcoreMesh(core_axis_name="c", subcore_axis_name="s"),
    out_type=jax.ShapeDtypeStruct((S, D), jnp.float32),
    compiler_params=pltpu.CompilerParams(needs_layout_passes=False),
    scratch_types=[
        pltpu.VMEM((S,), jnp.int32),
        pltpu.VMEM((NCHUNK, CSIZE), jnp.int32),
        pltpu.VMEM((CSIZE, D), jnp.float32),
        pltpu.SemaphoreType.DMA,
        pltpu.SemaphoreType.DMA,
    ],
)(_body)


def kernel(x, masks):
    out = _routed(x[0], masks[0].astype(jnp.int32))
    return out[None]
